# baseline (speedup 1.0000x reference)
import jax
import jax.numpy as jnp
from jax import lax
from jax.experimental import pallas as pl
from jax.experimental.pallas import tpu as pltpu
from jax.experimental.pallas import tpu_sc as plsc

N = 1000000
D = 64
DIM = 100000
HALF = DIM // 2
NS = 16
EV = N // NS
EV_PAD = 62504
TBUF = 62512
W = 4000
NWIN = 15
TAIL = 2504
TAIL_CHUNKS = 157
TMP = 2000
BLK = 2000
NBLK = HALF // BLK
G = 128
NG = BLK // G
IDS = 2048
NBLK2 = DIM // BLK


def _tile_lo(s):
    return (s * EV) // 8 * 8


def _vgather(v, idx):
    dnums = lax.GatherDimensionNumbers(
        offset_dims=(), collapsed_slice_dims=(0,), start_index_map=(0,))
    return lax.gather(v, idx[:, None], dnums, (1,),
                      mode=lax.GatherScatterMode.PROMISE_IN_BOUNDS)


def _k1_body(idx_hbm, t_hbm, ids_hbm, arg_sp, t_sp):
    c = lax.axis_index("c")
    s = lax.axis_index("s")
    lo = _tile_lo(s)
    hi = _tile_lo(s + 1)
    seg_base = c * HALF
    stage_row = c * NS + s
    iota = lax.iota(jnp.int32, 16)

    def phase1(arg_tbl, t_chunk, idxw, tmp):
        neg1 = jnp.full((16,), -1, jnp.int32)

        def init_body(i, _):
            arg_tbl[pl.ds(i * 16, 16)] = neg1
            return 0
        lax.fori_loop(0, HALF // 16, init_body, 0)

        pltpu.sync_copy(t_hbm.at[pl.ds(lo, EV_PAD)], t_chunk.at[pl.ds(0, EV_PAD)])

        def chunk_body(local_base, idx_vec):
            i_vec = lo + local_base + iota
            off = idx_vec - seg_base
            active = (i_vec < hi) & (off >= 0) & (off < HALF)
            t_vec = t_chunk[pl.ds(local_base, 16)]

            def one_round(act):
                offs = jnp.where(act, off, 0)
                cur_a = plsc.load_gather(arg_tbl, [offs])
                hasc = act & (cur_a >= 0)
                lp = jnp.where(hasc, cur_a - lo, 0)
                cur_t = plsc.load_gather(t_chunk, [lp])
                beat = (~hasc) | (t_vec > cur_t) | ((t_vec == cur_t) & (i_vec > cur_a))
                win = act & beat
                plsc.store_scatter(arg_tbl, [offs], i_vec, mask=win)
                a2 = plsc.load_gather(arg_tbl, [offs])
                return win & (a2 != i_vec)

            lost = one_round(active)

            @pl.when(jnp.any(lost))
            def _():
                bit = jnp.int32(1) << iota

                def rb(_, mbits):
                    act = ((mbits >> iota) & 1) != 0
                    l2 = one_round(act)
                    return jnp.sum(jnp.where(l2, bit, 0))
                lax.fori_loop(0, 15, rb,
                              jnp.sum(jnp.where(lost, bit, 0)))

        def win_body(w, _):
            pltpu.sync_copy(idx_hbm.at[pl.ds(lo + w * W, W)], idxw)

            def cb(k, _):
                chunk_body(w * W + k * 16, idxw[pl.ds(k * 16, 16)])
                return 0
            lax.fori_loop(0, W // 16, cb, 0)
            return 0
        lax.fori_loop(0, NWIN, win_body, 0)

        pltpu.sync_copy(idx_hbm.at[pl.ds(lo + NWIN * W, TAIL)],
                        idxw.at[pl.ds(0, TAIL)])

        def cbt(k, _):
            chunk_body(NWIN * W + k * 16, idxw[pl.ds(k * 16, 16)])
            return 0
        lax.fori_loop(0, TAIL_CHUNKS, cbt, 0)

        pltpu.sync_copy(arg_tbl, arg_sp.at[stage_row])

        def stage_body(it, _):
            def g(gi, _):
                a = arg_tbl[pl.ds(it * TMP + gi * 16, 16)]
                hasv = a >= 0
                lp = jnp.where(hasv, a - lo, 0)
                tv = plsc.load_gather(t_chunk, [lp])
                tmp[pl.ds(gi * 16, 16)] = jnp.where(hasv, tv, jnp.float32(-1.0))
                return 0
            lax.fori_loop(0, TMP // 16, g, 0)
            pltpu.sync_copy(tmp, t_sp.at[stage_row, pl.ds(it * TMP, TMP)])
            return 0
        lax.fori_loop(0, HALF // TMP, stage_body, 0)

    pl.run_scoped(
        phase1,
        pltpu.VMEM((HALF,), jnp.int32),
        pltpu.VMEM((TBUF,), jnp.float32),
        pltpu.VMEM((W,), jnp.int32),
        pltpu.VMEM((TMP,), jnp.float32),
    )

    plsc.subcore_barrier()

    def phase2(marg, mt, ids_buf):
        def do_block(b):
            pltpu.sync_copy(
                arg_sp.at[pl.ds(c * NS, NS), pl.ds(b * BLK, BLK)], marg)
            pltpu.sync_copy(
                t_sp.at[pl.ds(c * NS, NS), pl.ds(b * BLK, BLK)], mt)

            def mg(v, _):
                best_t = mt[0, pl.ds(v * 16, 16)]
                best_a = marg[0, pl.ds(v * 16, 16)]
                for j in range(1, NS):
                    ta = mt[j, pl.ds(v * 16, 16)]
                    aa = marg[j, pl.ds(v * 16, 16)]
                    upd = (ta > best_t) | ((ta == best_t) & (aa > best_a))
                    best_t = jnp.where(upd, ta, best_t)
                    best_a = jnp.where(upd, aa, best_a)
                ids_buf[pl.ds(v * 16, 16)] = best_a
                return 0
            lax.fori_loop(0, BLK // 16, mg, 0)

            pltpu.sync_copy(ids_buf,
                            ids_hbm.at[pl.ds(seg_base + b * BLK, BLK)])

        def blk_loop(k, _):
            b = s + k * NS

            @pl.when(b < NBLK)
            def _():
                do_block(b)
            return 0
        lax.fori_loop(0, (NBLK + NS - 1) // NS, blk_loop, 0)

    pl.run_scoped(
        phase2,
        pltpu.VMEM((NS, BLK), jnp.int32),
        pltpu.VMEM((NS, BLK), jnp.float32),
        pltpu.VMEM((BLK,), jnp.int32),
    )


def _k2_body(msg2_hbm, ids_hbm, out2_hbm, sem):
    c = lax.axis_index("c")
    s = lax.axis_index("s")
    wid = c * NS + s
    iota = lax.iota(jnp.int32, 16)

    def gather_phase(ids_buf, bid_buf, vld_buf, rows2, stage):
        def do_block(b):
            pltpu.sync_copy(ids_hbm.at[pl.ds(b * BLK, BLK)],
                            ids_buf.at[pl.ds(0, BLK)])

            def clamp(v, _):
                a = ids_buf[pl.ds(v * 16, 16)]
                valid = a >= 0
                safe = jnp.where(valid, a, 0)
                ids_buf[pl.ds(v * 16, 16)] = safe
                bid_buf[pl.ds(v * 16, 16)] = safe >> 1
                vld_buf[pl.ds(v * 16, 16)] = jnp.where(valid, 1, 0)
                return 0
            lax.fori_loop(0, BLK // 16, clamp, 0)

            zero16 = jnp.zeros((16,), jnp.int32)
            one16 = jnp.full((16,), 1, jnp.int32)
            for p in range((IDS - BLK) // 16):
                ids_buf[pl.ds(BLK + p * 16, 16)] = zero16
                bid_buf[pl.ds(BLK + p * 16, 16)] = zero16
                vld_buf[pl.ds(BLK + p * 16, 16)] = one16

            def gather_group(g, nrows):
                pltpu.async_copy(
                    msg2_hbm.at[bid_buf.at[pl.ds(g * G, G)]], rows2, sem
                ).wait()

                def sel(sub, _):
                    ids16 = ids_buf[pl.ds(g * G + sub * 16, 16)]
                    hvec = (ids16 & 1) * 64
                    for l in range(16):
                        hl = _vgather(hvec, jnp.full((16,), l, jnp.int32))
                        for q in range(4):
                            col = hl + (q * 16 + iota)
                            row = sub * 16 + l
                            w = plsc.load_gather(
                                rows2, [jnp.full((16,), row, jnp.int32), col])
                            stage[row // 2, pl.ds((row % 2) * 64 + q * 16, 16)] = w
                    return 0
                lax.fori_loop(0, G // 16, sel, 0)

                def fix_h(h, _):
                    bad = vld_buf[pl.ds(g * G + h * 16, 16)] == 0

                    @pl.when(jnp.any(bad))
                    def _():
                        def fix_lane(l, _):
                            @pl.when(jnp.any((iota == l) & bad))
                            def _():
                                z = jnp.zeros((16,), jnp.float32)
                                r = h * 16 + l
                                for q in range(4):
                                    stage[r // 2,
                                          pl.ds((r % 2) * 64 + q * 16, 16)] = z
                            return 0
                        lax.fori_loop(0, 16, fix_lane, 0)
                    return 0
                lax.fori_loop(0, G // 16, fix_h, 0)

                row0 = b * BLK + g * G
                pltpu.sync_copy(stage.at[pl.ds(0, nrows // 2)],
                                out2_hbm.at[pl.ds(row0 // 2, nrows // 2)])

            def gg(g, _):
                gather_group(g, G)
                return 0
            lax.fori_loop(0, NG, gg, 0)
            gather_group(NG, BLK - NG * G)

        def blk_loop(k, _):
            b = wid + k * 2 * NS

            @pl.when(b < NBLK2)
            def _():
                do_block(b)
            return 0
        lax.fori_loop(0, (NBLK2 + 2 * NS - 1) // (2 * NS), blk_loop, 0)

    pl.run_scoped(
        gather_phase,
        pltpu.VMEM((IDS,), jnp.int32),
        pltpu.VMEM((IDS,), jnp.int32),
        pltpu.VMEM((IDS,), jnp.int32),
        pltpu.VMEM((G, 2 * D), jnp.float32),
        pltpu.VMEM((G // 2, 2 * D), jnp.float32),
    )


def kernel(msg, index, t, dim_size, args):
    mesh = plsc.VectorSubcoreMesh(core_axis_name="c", subcore_axis_name="s")
    params = pltpu.CompilerParams(
        needs_layout_passes=False, use_tc_tiling_on_sc=False)
    k1 = pl.kernel(
        _k1_body,
        out_type=jax.ShapeDtypeStruct((DIM,), jnp.int32),
        mesh=mesh,
        compiler_params=params,
        scratch_types=[
            pltpu.HBM((2 * NS, HALF), jnp.int32),
            pltpu.HBM((2 * NS, HALF), jnp.float32),
        ],
    )
    k2 = pl.kernel(
        _k2_body,
        out_type=jax.ShapeDtypeStruct((DIM // 2, 2 * D), jnp.float32),
        mesh=mesh,
        compiler_params=params,
        scratch_types=[pltpu.SemaphoreType.DMA],
    )
    ids = k1(index, t)
    out2 = k2(jnp.reshape(msg, (N // 2, 2 * D)), ids)
    return jnp.reshape(out2, (DIM, D))


# speedup vs baseline: 1.1995x; 1.1995x over previous
import jax
import jax.numpy as jnp
from jax import lax
from jax.experimental import pallas as pl
from jax.experimental.pallas import tpu as pltpu
from jax.experimental.pallas import tpu_sc as plsc

N = 1000000
D = 64
DIM = 100000
HALF = DIM // 2
NS = 16
EV = N // NS
EV_PAD = 62504
TBUF = 62512
W = 4000
NWIN = 15
TAIL = 2504
TAIL_CHUNKS = 157
TMP = 2000
BLK = 2000
NBLK = HALF // BLK
G = 128
NG = BLK // G
IDS = 2048
NBLK2 = DIM // BLK


def _tile_lo(s):
    return (s * EV) // 8 * 8


def _k1_body(idx_hbm, t_hbm, ids_hbm, arg_sp, t_sp):
    c = lax.axis_index("c")
    s = lax.axis_index("s")
    lo = _tile_lo(s)
    hi = _tile_lo(s + 1)
    seg_base = c * HALF
    stage_row = c * NS + s
    iota = lax.iota(jnp.int32, 16)

    def phase1(arg_tbl, t_chunk, idxw, tmp):
        neg1 = jnp.full((16,), -1, jnp.int32)

        def init_body(i, _):
            arg_tbl[pl.ds(i * 16, 16)] = neg1
            return 0
        lax.fori_loop(0, HALF // 16, init_body, 0)

        pltpu.sync_copy(t_hbm.at[pl.ds(lo, EV_PAD)], t_chunk.at[pl.ds(0, EV_PAD)])

        def chunk_body(local_base, idx_vec):
            i_vec = lo + local_base + iota
            off = idx_vec - seg_base
            active = (i_vec < hi) & (off >= 0) & (off < HALF)
            t_vec = t_chunk[pl.ds(local_base, 16)]

            def one_round(act):
                offs = jnp.where(act, off, 0)
                cur_a = plsc.load_gather(arg_tbl, [offs])
                hasc = act & (cur_a >= 0)
                lp = jnp.where(hasc, cur_a - lo, 0)
                cur_t = plsc.load_gather(t_chunk, [lp])
                beat = (~hasc) | (t_vec > cur_t) | ((t_vec == cur_t) & (i_vec > cur_a))
                win = act & beat
                plsc.store_scatter(arg_tbl, [offs], i_vec, mask=win)
                a2 = plsc.load_gather(arg_tbl, [offs])
                return win & (a2 != i_vec)

            lost = one_round(active)

            @pl.when(jnp.any(lost))
            def _():
                bit = jnp.int32(1) << iota

                def rb(_, mbits):
                    act = ((mbits >> iota) & 1) != 0
                    l2 = one_round(act)
                    return jnp.sum(jnp.where(l2, bit, 0))
                lax.fori_loop(0, 15, rb,
                              jnp.sum(jnp.where(lost, bit, 0)))

        def win_body(w, _):
            pltpu.sync_copy(idx_hbm.at[pl.ds(lo + w * W, W)], idxw)

            def cb(k, _):
                chunk_body(w * W + k * 16, idxw[pl.ds(k * 16, 16)])
                return 0
            lax.fori_loop(0, W // 16, cb, 0)
            return 0
        lax.fori_loop(0, NWIN, win_body, 0)

        pltpu.sync_copy(idx_hbm.at[pl.ds(lo + NWIN * W, TAIL)],
                        idxw.at[pl.ds(0, TAIL)])

        def cbt(k, _):
            chunk_body(NWIN * W + k * 16, idxw[pl.ds(k * 16, 16)])
            return 0
        lax.fori_loop(0, TAIL_CHUNKS, cbt, 0)

        pltpu.sync_copy(arg_tbl, arg_sp.at[stage_row])

        def stage_body(it, _):
            def g(gi, _):
                a = arg_tbl[pl.ds(it * TMP + gi * 16, 16)]
                hasv = a >= 0
                lp = jnp.where(hasv, a - lo, 0)
                tv = plsc.load_gather(t_chunk, [lp])
                tmp[pl.ds(gi * 16, 16)] = jnp.where(hasv, tv, jnp.float32(-1.0))
                return 0
            lax.fori_loop(0, TMP // 16, g, 0)
            pltpu.sync_copy(tmp, t_sp.at[stage_row, pl.ds(it * TMP, TMP)])
            return 0
        lax.fori_loop(0, HALF // TMP, stage_body, 0)

    pl.run_scoped(
        phase1,
        pltpu.VMEM((HALF,), jnp.int32),
        pltpu.VMEM((TBUF,), jnp.float32),
        pltpu.VMEM((W,), jnp.int32),
        pltpu.VMEM((TMP,), jnp.float32),
    )

    plsc.subcore_barrier()

    def phase2(marg, mt, ids_buf):
        def do_block(b):
            pltpu.sync_copy(
                arg_sp.at[pl.ds(c * NS, NS), pl.ds(b * BLK, BLK)], marg)
            pltpu.sync_copy(
                t_sp.at[pl.ds(c * NS, NS), pl.ds(b * BLK, BLK)], mt)

            def mg(v, _):
                best_t = mt[0, pl.ds(v * 16, 16)]
                best_a = marg[0, pl.ds(v * 16, 16)]
                for j in range(1, NS):
                    ta = mt[j, pl.ds(v * 16, 16)]
                    aa = marg[j, pl.ds(v * 16, 16)]
                    upd = (ta > best_t) | ((ta == best_t) & (aa > best_a))
                    best_t = jnp.where(upd, ta, best_t)
                    best_a = jnp.where(upd, aa, best_a)
                ids_buf[pl.ds(v * 16, 16)] = best_a
                return 0
            lax.fori_loop(0, BLK // 16, mg, 0)

            pltpu.sync_copy(ids_buf,
                            ids_hbm.at[pl.ds(seg_base + b * BLK, BLK)])

        def blk_loop(k, _):
            b = s + k * NS

            @pl.when(b < NBLK)
            def _():
                do_block(b)
            return 0
        lax.fori_loop(0, (NBLK + NS - 1) // NS, blk_loop, 0)

    pl.run_scoped(
        phase2,
        pltpu.VMEM((NS, BLK), jnp.int32),
        pltpu.VMEM((NS, BLK), jnp.float32),
        pltpu.VMEM((BLK,), jnp.int32),
    )


def _k2_body(msg_hbm, ids_hbm, out_hbm, sem):
    c = lax.axis_index("c")
    s = lax.axis_index("s")
    wid = c * NS + s
    iota = lax.iota(jnp.int32, 16)

    def gather_phase(ids_buf, vld_buf, rows):
        def do_block(b):
            pltpu.sync_copy(ids_hbm.at[pl.ds(b * BLK, BLK)],
                            ids_buf.at[pl.ds(0, BLK)])

            def clamp(v, _):
                a = ids_buf[pl.ds(v * 16, 16)]
                valid = a >= 0
                ids_buf[pl.ds(v * 16, 16)] = jnp.where(valid, a, 0)
                vld_buf[pl.ds(v * 16, 16)] = jnp.where(valid, 1, 0)
                return 0
            lax.fori_loop(0, BLK // 16, clamp, 0)

            zero16 = jnp.zeros((16,), jnp.int32)
            one16 = jnp.full((16,), 1, jnp.int32)
            for p in range((IDS - BLK) // 16):
                ids_buf[pl.ds(BLK + p * 16, 16)] = zero16
                vld_buf[pl.ds(BLK + p * 16, 16)] = one16

            def gather_group(g, nrows):
                pltpu.async_copy(
                    msg_hbm.at[ids_buf.at[pl.ds(g * G, G)]], rows, sem
                ).wait()

                def fix_h(h, _):
                    bad = vld_buf[pl.ds(g * G + h * 16, 16)] == 0

                    @pl.when(jnp.any(bad))
                    def _():
                        def fix_lane(l, _):
                            @pl.when(jnp.any((iota == l) & bad))
                            def _():
                                z = jnp.zeros((16,), jnp.float32)
                                r = h * 16 + l
                                for q in range(4):
                                    rows[r, pl.ds(q * 16, 16)] = z
                            return 0
                        lax.fori_loop(0, 16, fix_lane, 0)
                    return 0
                lax.fori_loop(0, G // 16, fix_h, 0)

                row0 = b * BLK + g * G
                pltpu.sync_copy(rows.at[pl.ds(0, nrows)],
                                out_hbm.at[pl.ds(row0, nrows)])

            def gg(g, _):
                gather_group(g, G)
                return 0
            lax.fori_loop(0, NG, gg, 0)
            gather_group(NG, BLK - NG * G)

        def blk_loop(k, _):
            b = wid + k * 2 * NS

            @pl.when(b < NBLK2)
            def _():
                do_block(b)
            return 0
        lax.fori_loop(0, (NBLK2 + 2 * NS - 1) // (2 * NS), blk_loop, 0)

    pl.run_scoped(
        gather_phase,
        pltpu.VMEM((IDS,), jnp.int32),
        pltpu.VMEM((IDS,), jnp.int32),
        pltpu.VMEM((G, D), jnp.float32),
    )


def kernel(msg, index, t, dim_size, args):
    mesh = plsc.VectorSubcoreMesh(core_axis_name="c", subcore_axis_name="s")
    params = pltpu.CompilerParams(
        needs_layout_passes=False, use_tc_tiling_on_sc=False)
    k1 = pl.kernel(
        _k1_body,
        out_type=jax.ShapeDtypeStruct((DIM,), jnp.int32),
        mesh=mesh,
        compiler_params=params,
        scratch_types=[
            pltpu.HBM((2 * NS, HALF), jnp.int32),
            pltpu.HBM((2 * NS, HALF), jnp.float32),
        ],
    )
    k2 = pl.kernel(
        _k2_body,
        out_type=jax.ShapeDtypeStruct((DIM, D), jnp.float32),
        mesh=mesh,
        compiler_params=params,
        scratch_types=[pltpu.SemaphoreType.DMA],
    )
    ids = k1(index, t)
    return k2(msg, ids)


# speedup vs baseline: 1.2247x; 1.0210x over previous
import jax
import jax.numpy as jnp
from jax import lax
from jax.experimental import pallas as pl
from jax.experimental.pallas import tpu as pltpu
from jax.experimental.pallas import tpu_sc as plsc

N = 1000000
D = 64
DIM = 100000
HALF = DIM // 2
NS = 16
EV = N // NS
EV_PAD = 62504
TBUF = 62512
W = 4000
NWIN = 15
TAIL = 2504
TAIL_CHUNKS = 157
TMP = 2000
BLK = 2000
NBLK = HALF // BLK
G = 128
NG = BLK // G
IDS = 2048
NBLK2 = DIM // BLK


def _tile_lo(s):
    return (s * EV) // 8 * 8


def _k1_body(idx_hbm, t_hbm, ids_hbm, arg_sp, t_sp):
    c = lax.axis_index("c")
    s = lax.axis_index("s")
    lo = _tile_lo(s)
    hi = _tile_lo(s + 1)
    seg_base = c * HALF
    stage_row = c * NS + s
    iota = lax.iota(jnp.int32, 16)

    def phase1(arg_tbl, t_chunk, idxw, tmp):
        neg1 = jnp.full((16,), -1, jnp.int32)

        def init_body(i, _):
            arg_tbl[pl.ds(i * 16, 16)] = neg1
            return 0
        lax.fori_loop(0, HALF // 16, init_body, 0)

        pltpu.sync_copy(t_hbm.at[pl.ds(lo, EV_PAD)], t_chunk.at[pl.ds(0, EV_PAD)])

        def chunk_body(local_base, idx_vec):
            i_vec = lo + local_base + iota
            off = idx_vec - seg_base
            active = (i_vec < hi) & (off >= 0) & (off < HALF)
            t_vec = t_chunk[pl.ds(local_base, 16)]

            def one_round(act):
                offs = jnp.where(act, off, 0)
                cur_a = plsc.load_gather(arg_tbl, [offs])
                hasc = act & (cur_a >= 0)
                lp = jnp.where(hasc, cur_a - lo, 0)
                cur_t = plsc.load_gather(t_chunk, [lp])
                beat = (~hasc) | (t_vec > cur_t) | ((t_vec == cur_t) & (i_vec > cur_a))
                win = act & beat
                plsc.store_scatter(arg_tbl, [offs], i_vec, mask=win)
                a2 = plsc.load_gather(arg_tbl, [offs])
                return win & (a2 != i_vec)

            lost = one_round(active)

            @pl.when(jnp.any(lost))
            def _():
                bit = jnp.int32(1) << iota

                def rb(_, mbits):
                    act = ((mbits >> iota) & 1) != 0
                    l2 = one_round(act)
                    return jnp.sum(jnp.where(l2, bit, 0))
                lax.fori_loop(0, 15, rb,
                              jnp.sum(jnp.where(lost, bit, 0)))

        def win_body(w, _):
            pltpu.sync_copy(idx_hbm.at[pl.ds(lo + w * W, W)], idxw)

            def cb(k, _):
                chunk_body(w * W + k * 16, idxw[pl.ds(k * 16, 16)])
                return 0
            lax.fori_loop(0, W // 16, cb, 0)
            return 0
        lax.fori_loop(0, NWIN, win_body, 0)

        pltpu.sync_copy(idx_hbm.at[pl.ds(lo + NWIN * W, TAIL)],
                        idxw.at[pl.ds(0, TAIL)])

        def cbt(k, _):
            chunk_body(NWIN * W + k * 16, idxw[pl.ds(k * 16, 16)])
            return 0
        lax.fori_loop(0, TAIL_CHUNKS, cbt, 0)

        pltpu.sync_copy(arg_tbl, arg_sp.at[stage_row])

        def stage_body(it, _):
            def g(gi, _):
                a = arg_tbl[pl.ds(it * TMP + gi * 16, 16)]
                hasv = a >= 0
                lp = jnp.where(hasv, a - lo, 0)
                tv = plsc.load_gather(t_chunk, [lp])
                tmp[pl.ds(gi * 16, 16)] = jnp.where(hasv, tv, jnp.float32(-1.0))
                return 0
            lax.fori_loop(0, TMP // 16, g, 0)
            pltpu.sync_copy(tmp, t_sp.at[stage_row, pl.ds(it * TMP, TMP)])
            return 0
        lax.fori_loop(0, HALF // TMP, stage_body, 0)

    pl.run_scoped(
        phase1,
        pltpu.VMEM((HALF,), jnp.int32),
        pltpu.VMEM((TBUF,), jnp.float32),
        pltpu.VMEM((W,), jnp.int32),
        pltpu.VMEM((TMP,), jnp.float32),
    )

    plsc.subcore_barrier()

    def phase2(marg, mt, ids_buf):
        def do_block(b):
            pltpu.sync_copy(
                arg_sp.at[pl.ds(c * NS, NS), pl.ds(b * BLK, BLK)], marg)
            pltpu.sync_copy(
                t_sp.at[pl.ds(c * NS, NS), pl.ds(b * BLK, BLK)], mt)

            def mg(v, _):
                best_t = mt[0, pl.ds(v * 16, 16)]
                best_a = marg[0, pl.ds(v * 16, 16)]
                for j in range(1, NS):
                    ta = mt[j, pl.ds(v * 16, 16)]
                    aa = marg[j, pl.ds(v * 16, 16)]
                    upd = (ta > best_t) | ((ta == best_t) & (aa > best_a))
                    best_t = jnp.where(upd, ta, best_t)
                    best_a = jnp.where(upd, aa, best_a)
                ids_buf[pl.ds(v * 16, 16)] = best_a
                return 0
            lax.fori_loop(0, BLK // 16, mg, 0)

            pltpu.sync_copy(ids_buf,
                            ids_hbm.at[pl.ds(seg_base + b * BLK, BLK)])

        def blk_loop(k, _):
            b = s + k * NS

            @pl.when(b < NBLK)
            def _():
                do_block(b)
            return 0
        lax.fori_loop(0, (NBLK + NS - 1) // NS, blk_loop, 0)

    pl.run_scoped(
        phase2,
        pltpu.VMEM((NS, BLK), jnp.int32),
        pltpu.VMEM((NS, BLK), jnp.float32),
        pltpu.VMEM((BLK,), jnp.int32),
    )


def _k2_body(msg_hbm, ids_hbm, out_hbm, sem_a, sem_b):
    c = lax.axis_index("c")
    s = lax.axis_index("s")
    wid = c * NS + s
    iota = lax.iota(jnp.int32, 16)
    NGRP = (BLK + G - 1) // G

    def gather_phase(ids_buf, vld_buf, rows_a, rows_b):
        def do_block(b):
            pltpu.sync_copy(ids_hbm.at[pl.ds(b * BLK, BLK)],
                            ids_buf.at[pl.ds(0, BLK)])

            def clamp(v, _):
                a = ids_buf[pl.ds(v * 16, 16)]
                valid = a >= 0
                ids_buf[pl.ds(v * 16, 16)] = jnp.where(valid, a, 0)
                vld_buf[pl.ds(v * 16, 16)] = jnp.where(valid, 1, 0)
                return 0
            lax.fori_loop(0, BLK // 16, clamp, 0)

            zero16 = jnp.zeros((16,), jnp.int32)
            one16 = jnp.full((16,), 1, jnp.int32)
            for p in range((IDS - BLK) // 16):
                ids_buf[pl.ds(BLK + p * 16, 16)] = zero16
                vld_buf[pl.ds(BLK + p * 16, 16)] = one16

            def issue(g, rows, sem):
                return pltpu.async_copy(
                    msg_hbm.at[ids_buf.at[pl.ds(g * G, G)]], rows, sem)

            def fix_write(g, rows, nrows):
                def fix_h(h, _):
                    bad = vld_buf[pl.ds(g * G + h * 16, 16)] == 0

                    @pl.when(jnp.any(bad))
                    def _():
                        def fix_lane(l, _):
                            @pl.when(jnp.any((iota == l) & bad))
                            def _():
                                z = jnp.zeros((16,), jnp.float32)
                                r = h * 16 + l
                                for q in range(4):
                                    rows[r, pl.ds(q * 16, 16)] = z
                            return 0
                        lax.fori_loop(0, 16, fix_lane, 0)
                    return 0
                lax.fori_loop(0, G // 16, fix_h, 0)

                row0 = b * BLK + g * G
                pltpu.sync_copy(rows.at[pl.ds(0, nrows)],
                                out_hbm.at[pl.ds(row0, nrows)])

            bufs = (rows_a, rows_b)
            sems = (sem_a, sem_b)
            descs = [None] * NGRP
            descs[0] = issue(0, bufs[0], sems[0])
            for g in range(NGRP):
                if g + 1 < NGRP:
                    descs[g + 1] = issue(g + 1, bufs[(g + 1) % 2],
                                         sems[(g + 1) % 2])
                descs[g].wait()
                fix_write(g, bufs[g % 2],
                          G if g < NGRP - 1 else BLK - (NGRP - 1) * G)

        def blk_loop(k, _):
            b = wid + k * 2 * NS

            @pl.when(b < NBLK2)
            def _():
                do_block(b)
            return 0
        lax.fori_loop(0, (NBLK2 + 2 * NS - 1) // (2 * NS), blk_loop, 0)

    pl.run_scoped(
        gather_phase,
        pltpu.VMEM((IDS,), jnp.int32),
        pltpu.VMEM((IDS,), jnp.int32),
        pltpu.VMEM((G, D), jnp.float32),
        pltpu.VMEM((G, D), jnp.float32),
    )


def kernel(msg, index, t, dim_size, args):
    mesh = plsc.VectorSubcoreMesh(core_axis_name="c", subcore_axis_name="s")
    params = pltpu.CompilerParams(
        needs_layout_passes=False, use_tc_tiling_on_sc=False)
    k1 = pl.kernel(
        _k1_body,
        out_type=jax.ShapeDtypeStruct((DIM,), jnp.int32),
        mesh=mesh,
        compiler_params=params,
        scratch_types=[
            pltpu.HBM((2 * NS, HALF), jnp.int32),
            pltpu.HBM((2 * NS, HALF), jnp.float32),
        ],
    )
    k2 = pl.kernel(
        _k2_body,
        out_type=jax.ShapeDtypeStruct((DIM, D), jnp.float32),
        mesh=mesh,
        compiler_params=params,
        scratch_types=[pltpu.SemaphoreType.DMA,
                       pltpu.SemaphoreType.DMA],
    )
    ids = k1(index, t)
    return k2(msg, ids)
